# Initial kernel scaffold; baseline (speedup 1.0000x reference)
import jax
import jax.numpy as jnp
from jax.experimental import pallas as pl


def kernel(index_map, colormap):
    raise NotImplementedError("write your pallas kernel here")


# speedup vs baseline: 70.6735x; 70.6735x over previous
import functools

import jax
import jax.numpy as jnp
from jax import lax
from jax.experimental import pallas as pl
from jax.experimental.pallas import tpu as pltpu
from jax.experimental.pallas import tpu_sc as plsc

B, H, W = 16, 512, 512
HW = H * W
N = B * HW
NC, NS, L = 2, 16, 16
NW = NC * NS
PX_PER_W = N // NW
CHUNK = 16384
NCHUNK = PX_PER_W // CHUNK
VPC = CHUNK // L


def _dye_body(idx_hbm, cmap_hbm, out_hbm, lut_v, idx_v, out_r, out_g, out_b):
  wid = lax.axis_index("s") * NC + lax.axis_index("c")
  b = wid // 2
  p_base = (wid % 2) * PX_PER_W

  pltpu.sync_copy(cmap_hbm, lut_v)

  for g in range(NCHUNK):
    p0 = p_base + g * CHUNK
    pltpu.sync_copy(idx_hbm.at[pl.ds(b * HW + p0, CHUNK)], idx_v)

    def vbody(i, _):
      sl = pl.ds(i * L, L)
      iv = idx_v[sl]
      for ch, buf in ((0, out_r), (1, out_g), (2, out_b)):
        buf[sl] = plsc.load_gather(lut_v, [iv + (ch * 256)])
      return 0

    lax.fori_loop(0, VPC, vbody, 0)

    for ch, buf in ((0, out_r), (1, out_g), (2, out_b)):
      pltpu.sync_copy(buf,
                      out_hbm.at[pl.ds((b * 3 + ch) * HW + p0, CHUNK)])


_dye = functools.partial(
    pl.kernel,
    out_type=jax.ShapeDtypeStruct((3 * N,), jnp.float32),
    mesh=plsc.VectorSubcoreMesh(core_axis_name="c", subcore_axis_name="s"),
    compiler_params=pltpu.CompilerParams(needs_layout_passes=False),
    scratch_types=[
        pltpu.VMEM((768,), jnp.float32),
        pltpu.VMEM((CHUNK,), jnp.int32),
        pltpu.VMEM((CHUNK,), jnp.float32),
        pltpu.VMEM((CHUNK,), jnp.float32),
        pltpu.VMEM((CHUNK,), jnp.float32),
    ],
)(_dye_body)


@jax.jit
def kernel(index_map, colormap):
  idx = index_map.astype(jnp.int32).reshape(N)
  cmap_t = colormap.T.reshape(768).astype(jnp.float32)
  out = _dye(idx, cmap_t)
  return out.reshape(B, 3, H, W)


# speedup vs baseline: 125.5923x; 1.7771x over previous
import functools

import jax
import jax.numpy as jnp
from jax import lax
from jax.experimental import pallas as pl
from jax.experimental.pallas import tpu as pltpu
from jax.experimental.pallas import tpu_sc as plsc

B, H, W = 16, 512, 512
HW = H * W
N = B * HW
NC, NS, L = 2, 16, 16
NW = NC * NS
PX_PER_W = N // NW
CHUNK = 8192
NCHUNK = PX_PER_W // CHUNK
VPC = CHUNK // L


def _dye_body(idx_hbm, cmap_hbm, out_hbm, lut_v,
              idx0, idx1, r0, g0, b0, r1, g1, b1,
              sem_in0, sem_in1, sem_out0, sem_out1):
  wid = lax.axis_index("s") * NC + lax.axis_index("c")
  b = wid // 2
  p_base = (wid % 2) * PX_PER_W

  pltpu.sync_copy(cmap_hbm, lut_v)

  slots = (
      (idx0, (r0, g0, b0), sem_in0, sem_out0),
      (idx1, (r1, g1, b1), sem_in1, sem_out1),
  )

  def start_in(g):
    idx_v, _, sem_in, _ = slots[g % 2]
    p0 = p_base + g * CHUNK
    return pltpu.async_copy(idx_hbm.at[pl.ds(b * HW + p0, CHUNK)], idx_v,
                            sem_in)

  for g in range(NCHUNK):
    s = g % 2
    idx_v, outs, _, sem_out = slots[s]
    p0 = p_base + g * CHUNK
    pltpu.sync_copy(idx_hbm.at[pl.ds(b * HW + p0, CHUNK)], idx_v)

    @functools.partial(plsc.parallel_loop, 0, VPC, unroll=8)
    def _(i):
      sl = pl.ds(i * L, L)
      iv = idx_v[sl]
      for ch, buf in enumerate(outs):
        buf[sl] = plsc.load_gather(lut_v, [iv + (ch * 256)])

    for ch, buf in enumerate(outs):
      pltpu.sync_copy(buf,
                      out_hbm.at[pl.ds((b * 3 + ch) * HW + p0, CHUNK)])


_dye = functools.partial(
    pl.kernel,
    out_type=jax.ShapeDtypeStruct((3 * N,), jnp.float32),
    mesh=plsc.VectorSubcoreMesh(core_axis_name="c", subcore_axis_name="s"),
    compiler_params=pltpu.CompilerParams(needs_layout_passes=False),
    scratch_types=[
        pltpu.VMEM((768,), jnp.float32),
        pltpu.VMEM((CHUNK,), jnp.int32),
        pltpu.VMEM((CHUNK,), jnp.int32),
        pltpu.VMEM((CHUNK,), jnp.float32),
        pltpu.VMEM((CHUNK,), jnp.float32),
        pltpu.VMEM((CHUNK,), jnp.float32),
        pltpu.VMEM((CHUNK,), jnp.float32),
        pltpu.VMEM((CHUNK,), jnp.float32),
        pltpu.VMEM((CHUNK,), jnp.float32),
        pltpu.SemaphoreType.DMA,
        pltpu.SemaphoreType.DMA,
        pltpu.SemaphoreType.DMA,
        pltpu.SemaphoreType.DMA,
    ],
)(_dye_body)


@jax.jit
def kernel(index_map, colormap):
  idx = index_map.astype(jnp.int32).reshape(N)
  cmap_t = colormap.T.reshape(768).astype(jnp.float32)
  out = _dye(idx, cmap_t)
  return out.reshape(B, 3, H, W)


# speedup vs baseline: 137.2773x; 1.0930x over previous
import functools

import jax
import jax.numpy as jnp
from jax import lax
from jax.experimental import pallas as pl
from jax.experimental.pallas import tpu as pltpu
from jax.experimental.pallas import tpu_sc as plsc

B, H, W = 16, 512, 512
HW = H * W
N = B * HW
NC, NS, L = 2, 16, 16
NW = NC * NS
PX_PER_W = N // NW
CHUNK = 8192
NCHUNK = PX_PER_W // CHUNK
VPC = CHUNK // L


def _dye_body(idx_hbm, cmap_hbm, out_hbm, lut_v,
              idx0, idx1, r0, g0, b0, r1, g1, b1,
              sem_in0, sem_in1, sem_out0, sem_out1):
  wid = lax.axis_index("s") * NC + lax.axis_index("c")
  b = wid // 2
  p_base = (wid % 2) * PX_PER_W

  pltpu.sync_copy(cmap_hbm, lut_v)

  slots = (
      (idx0, (r0, g0, b0), sem_in0, sem_out0),
      (idx1, (r1, g1, b1), sem_in1, sem_out1),
  )

  def start_in(g):
    idx_v, _, sem_in, _ = slots[g % 2]
    p0 = p_base + g * CHUNK
    return pltpu.async_copy(idx_hbm.at[pl.ds(b * HW + p0, CHUNK)], idx_v,
                            sem_in)

  pending_out = {0: [], 1: []}
  for g in range(NCHUNK):
    s = g % 2
    idx_v, outs, _, sem_out = slots[s]
    p0 = p_base + g * CHUNK
    pltpu.sync_copy(idx_hbm.at[pl.ds(b * HW + p0, CHUNK)], idx_v)
    for h in pending_out[s]:
      h.wait()
    pending_out[s] = []

    @functools.partial(plsc.parallel_loop, 0, VPC, unroll=8)
    def _(i):
      sl = pl.ds(i * L, L)
      iv = idx_v[sl]
      for ch, buf in enumerate(outs):
        buf[sl] = plsc.load_gather(lut_v, [iv + (ch * 256)])

    for ch, buf in enumerate(outs):
      pending_out[s].append(
          pltpu.async_copy(buf, out_hbm.at[pl.ds((b * 3 + ch) * HW + p0,
                                                 CHUNK)], sem_out))

  for s in (0, 1):
    for h in pending_out[s]:
      h.wait()


_dye = functools.partial(
    pl.kernel,
    out_type=jax.ShapeDtypeStruct((3 * N,), jnp.float32),
    mesh=plsc.VectorSubcoreMesh(core_axis_name="c", subcore_axis_name="s"),
    compiler_params=pltpu.CompilerParams(needs_layout_passes=False),
    scratch_types=[
        pltpu.VMEM((768,), jnp.float32),
        pltpu.VMEM((CHUNK,), jnp.int32),
        pltpu.VMEM((CHUNK,), jnp.int32),
        pltpu.VMEM((CHUNK,), jnp.float32),
        pltpu.VMEM((CHUNK,), jnp.float32),
        pltpu.VMEM((CHUNK,), jnp.float32),
        pltpu.VMEM((CHUNK,), jnp.float32),
        pltpu.VMEM((CHUNK,), jnp.float32),
        pltpu.VMEM((CHUNK,), jnp.float32),
        pltpu.SemaphoreType.DMA,
        pltpu.SemaphoreType.DMA,
        pltpu.SemaphoreType.DMA,
        pltpu.SemaphoreType.DMA,
    ],
)(_dye_body)


@jax.jit
def kernel(index_map, colormap):
  idx = index_map.astype(jnp.int32).reshape(N)
  cmap_t = colormap.T.reshape(768).astype(jnp.float32)
  out = _dye(idx, cmap_t)
  return out.reshape(B, 3, H, W)
